# baseline (speedup 1.0000x reference)
import jax
import jax.numpy as jnp
from jax import lax
from jax.experimental import pallas as pl
from jax.experimental.pallas import tpu as pltpu
from jax.experimental.pallas import tpu_sc as plsc

_NC = 2
_NS = 16
_NW = _NC * _NS

_CHUNK = 128
_NBUF = 2
_L = 16


def _emb_body(item_idx, user_idx, t2, t2u, out1, out1u,
              idx_v, pidx_v, rows_v, comp_v,
              uidx_v, upidx_v, urows_v, ucomp_v,
              gsem, ssem, osem, usem):
    wid = lax.axis_index("s") * _NC + lax.axis_index("c")
    per_w = idx_v.shape[0]
    n_chunks = per_w // _CHUNK
    base_w = wid * per_w

    pltpu.sync_copy(item_idx.at[pl.ds(base_w, per_w)], idx_v)

    def mk_pidx(v, _):
        x = idx_v[pl.ds(v * _L, _L)]
        pidx_v[pl.ds(v * _L, _L)] = lax.shift_right_logical(x, 1)
        return _

    lax.fori_loop(0, per_w // _L, mk_pidx, 0)

    per_w_user = uidx_v.shape[0]
    pltpu.sync_copy(user_idx.at[pl.ds(wid * per_w_user, per_w_user)], uidx_v)

    def mk_upidx(v, _):
        x = uidx_v[pl.ds(v * _L, _L)]
        upidx_v[pl.ds(v * _L, _L)] = lax.shift_right_logical(x, 3)
        return _

    lax.fori_loop(0, per_w_user // _L, mk_upidx, 0)
    pltpu.async_copy(t2u.at[upidx_v], urows_v, usem)

    def gather(j, b):
        return pltpu.make_async_copy(
            t2.at[pidx_v.at[pl.ds(j * _CHUNK, _CHUNK)]],
            rows_v.at[b],
            gsem.at[b],
        )

    def select_chunk(j, b):
        def one16(v, _):
            offs = (idx_v[pl.ds(j * _CHUNK + v * _L, _L)] & 1) * 64
            for i in range(_L):
                kk = v * _L + i
                for q in range(4):
                    comp_v[b, lax.div(kk, 2), pl.ds(lax.rem(kk, 2) * 64 + q * _L, _L)] = (
                        rows_v[b, kk, pl.ds(offs[i] + q * _L, _L)]
                    )
            return _

        lax.fori_loop(0, _CHUNK // _L, one16, 0)

    def store(j, b):
        row0 = pl.multiple_of((base_w + j * _CHUNK) // 2, 64)
        return pltpu.make_async_copy(
            comp_v.at[b],
            out1.at[pl.ds(row0, _CHUNK // 2)],
            osem.at[b],
        )

    for b in range(_NBUF):
        gather(b, b).start()

    def lap_body(lap, carry):
        jj = lap * _NBUF
        for b in range(_NBUF):
            gather(jj + b, b).wait()
            select_chunk(jj + b, b)
            store(jj + b, b).start()
        for b in range(_NBUF):
            store(jj + b, b).wait()
            gather(jj + _NBUF + b, b).start()
        return carry

    lax.fori_loop(0, n_chunks // _NBUF - 1, lap_body, 0)

    jj = n_chunks - _NBUF
    for b in range(_NBUF):
        gather(jj + b, b).wait()
        select_chunk(jj + b, b)
        store(jj + b, b).start()
    for b in range(_NBUF):
        store(jj + b, b).wait()

    pltpu.make_async_copy(t2u.at[upidx_v], urows_v, usem).wait()

    def uone16(v, _):
        offs = (uidx_v[pl.ds(v * _L, _L)] & 7) * _L
        for i in range(_L):
            kk = v * _L + i
            ucomp_v[lax.div(kk, 8), pl.ds(lax.rem(kk, 8) * _L, _L)] = (
                urows_v[kk, pl.ds(offs[i], _L)]
            )
        return _

    lax.fori_loop(0, per_w_user // _L, uone16, 0)
    urow0 = pl.multiple_of(wid * (per_w_user // 8), 16)
    pltpu.sync_copy(ucomp_v, out1u.at[pl.ds(urow0, per_w_user // 8)])


def kernel(values_item_hist, values_user_cat, table_item, table_user):
    n_hist = values_item_hist.shape[0]
    n_user = values_user_cat.shape[0]
    dim_item = table_item.shape[1]
    dim_user = table_user.shape[1]
    vocab_item = table_item.shape[0]
    vocab_user = table_user.shape[0]

    per_w = n_hist // _NW
    per_w_user = n_user // _NW
    pack_i = 128 // dim_item
    pack_u = 128 // dim_user

    t2 = table_item.reshape(vocab_item // pack_i, 128)
    t2u = table_user.reshape(vocab_user // pack_u, 128)

    mesh = plsc.VectorSubcoreMesh(core_axis_name="c", subcore_axis_name="s")
    f = pl.kernel(
        _emb_body,
        out_type=(
            jax.ShapeDtypeStruct((n_hist // pack_i, 128), jnp.float32),
            jax.ShapeDtypeStruct((n_user // pack_u, 128), jnp.float32),
        ),
        mesh=mesh,
        scratch_types=[
            pltpu.VMEM((per_w,), jnp.int32),
            pltpu.VMEM((per_w,), jnp.int32),
            pltpu.VMEM((_NBUF, _CHUNK, 128), jnp.float32),
            pltpu.VMEM((_NBUF, _CHUNK // 2, 128), jnp.float32),
            pltpu.VMEM((per_w_user,), jnp.int32),
            pltpu.VMEM((per_w_user,), jnp.int32),
            pltpu.VMEM((per_w_user, 128), jnp.float32),
            pltpu.VMEM((per_w_user // 8, 128), jnp.float32),
            pltpu.SemaphoreType.DMA((_NBUF,)),
            pltpu.SemaphoreType.DMA,
            pltpu.SemaphoreType.DMA((_NBUF,)),
            pltpu.SemaphoreType.DMA,
        ],
    )
    out1, out1u = f(values_item_hist, values_user_cat, t2, t2u)
    return (out1.reshape(n_hist, dim_item), out1u.reshape(n_user, dim_user))
